# baseline (speedup 1.0000x reference)
import jax
import jax.numpy as jnp
from jax.experimental import pallas as pl
from jax.experimental.pallas import tpu as pltpu

T, D_IN, D_OUT, E, RANK, TOP_K = 4096, 1024, 1024, 8, 16, 2
TILE = 1024


def _fused_kernel(x_ref, wb_ref, bb_ref, wr_ref, af_ref, bf_ref, out_ref):
    x = x_ref[...]
    xh = x.astype(jnp.bfloat16)

    lT = jax.lax.dot_general(
        wr_ref[...], x, (((1,), (1,)), ((), ())),
        preferred_element_type=jnp.float32)
    m = jnp.max(lT, axis=0, keepdims=True)
    eg = jnp.exp(lT - m)
    s_all = jnp.sum(eg, axis=0, keepdims=True)

    eidx = jax.lax.broadcasted_iota(jnp.int32, (E, TILE), 0)
    i1 = jnp.min(jnp.where(lT == m, eidx, E), axis=0, keepdims=True)
    mask1 = eidx == i1
    l2 = jnp.where(mask1, float("-inf"), lT)
    m2 = jnp.max(l2, axis=0, keepdims=True)
    i2 = jnp.min(jnp.where(l2 == m2, eidx, E), axis=0, keepdims=True)
    mask = mask1 | (eidx == i2)

    egm = jnp.where(mask, eg, 0.0)
    s_top = jnp.sum(egm, axis=0, keepdims=True)
    gate_nT = (egm / (s_top + 1e-6 * s_all)).astype(jnp.bfloat16)

    re = jax.lax.broadcasted_iota(jnp.int32, (E, E * RANK), 0)
    rc = jax.lax.broadcasted_iota(jnp.int32, (E, E * RANK), 1)
    rep = (rc // RANK == re).astype(jnp.bfloat16)
    gate_rep = jax.lax.dot_general(
        gate_nT, rep, (((0,), (0,)), ((), ())),
        preferred_element_type=jnp.float32)

    h = jax.lax.dot_general(
        xh, af_ref[...], (((1,), (1,)), ((), ())),
        preferred_element_type=jnp.float32)
    hw = (h * gate_rep).astype(jnp.bfloat16)

    CH = D_OUT // 4
    for c in range(4):
        sl = pl.ds(c * CH, CH)
        base_c = jax.lax.dot_general(
            xh, wb_ref[sl, :], (((1,), (1,)), ((), ())),
            preferred_element_type=jnp.float32)
        lora_c = jax.lax.dot_general(
            hw, bf_ref[:, sl], (((1,), (0,)), ((), ())),
            preferred_element_type=jnp.float32)
        out_ref[:, sl] = base_c + bb_ref[:, sl] + lora_c


def kernel(x, W_base, b_base, W_router, A, B):
    wbh = W_base.astype(jnp.bfloat16)
    af = A.reshape(E * RANK, D_IN).astype(jnp.bfloat16)
    bf = jnp.transpose(B, (0, 2, 1)).reshape(E * RANK, D_OUT).astype(jnp.bfloat16)
    bb = b_base.reshape(1, D_OUT)

    grid = (T // TILE,)
    return pl.pallas_call(
        _fused_kernel,
        grid=grid,
        in_specs=[
            pl.BlockSpec((TILE, D_IN), lambda i: (i, 0)),
            pl.BlockSpec((D_OUT, D_IN), lambda i: (0, 0)),
            pl.BlockSpec((1, D_OUT), lambda i: (0, 0)),
            pl.BlockSpec((E, D_IN), lambda i: (0, 0)),
            pl.BlockSpec((E * RANK, D_IN), lambda i: (0, 0)),
            pl.BlockSpec((E * RANK, D_OUT), lambda i: (0, 0)),
        ],
        out_specs=pl.BlockSpec((TILE, D_OUT), lambda i: (i, 0)),
        out_shape=jax.ShapeDtypeStruct((T, D_OUT), jnp.float32),
        compiler_params=pltpu.CompilerParams(
            dimension_semantics=("parallel",),
        ),
    )(x, wbh, bb, W_router, af, bf)


# speedup vs baseline: 1.1333x; 1.1333x over previous
import jax
import jax.numpy as jnp
from jax.experimental import pallas as pl
from jax.experimental.pallas import tpu as pltpu

T, D_IN, D_OUT, E, RANK, TOP_K = 4096, 1024, 1024, 8, 16, 2
TILE = 1024


def _fused_kernel(x_ref, wb_ref, bb_ref, wr_ref, abf_ref, out_ref,
                  wb_s):
    @pl.when(pl.program_id(0) == 0)
    def _prep():
        wb_s[...] = wb_ref[...].astype(jnp.bfloat16)

    x = x_ref[...]
    xh = x.astype(jnp.bfloat16)

    lT = jax.lax.dot_general(
        wr_ref[...], x, (((1,), (1,)), ((), ())),
        preferred_element_type=jnp.float32)
    m = jnp.max(lT, axis=0, keepdims=True)
    eg = jnp.exp(lT - m)
    s_all = jnp.sum(eg, axis=0, keepdims=True)

    eidx = jax.lax.broadcasted_iota(jnp.int32, (E, TILE), 0)
    i1 = jnp.min(jnp.where(lT == m, eidx, E), axis=0, keepdims=True)
    mask1 = eidx == i1
    l2 = jnp.where(mask1, float("-inf"), lT)
    m2 = jnp.max(l2, axis=0, keepdims=True)
    i2 = jnp.min(jnp.where(l2 == m2, eidx, E), axis=0, keepdims=True)
    mask = mask1 | (eidx == i2)

    egm = jnp.where(mask, eg, 0.0)
    s_top = jnp.sum(egm, axis=0, keepdims=True)
    gate_nT = (egm / (s_top + 1e-6 * s_all)).astype(jnp.bfloat16)

    re = jax.lax.broadcasted_iota(jnp.int32, (E, E * RANK), 0)
    rc = jax.lax.broadcasted_iota(jnp.int32, (E, E * RANK), 1)
    rep = (rc // RANK == re).astype(jnp.bfloat16)
    gate_rep = jax.lax.dot_general(
        gate_nT, rep, (((0,), (0,)), ((), ())),
        preferred_element_type=jnp.float32)

    h = jax.lax.dot_general(
        xh, abf_ref[:, :D_IN], (((1,), (1,)), ((), ())),
        preferred_element_type=jnp.float32)
    hw = (h * gate_rep).astype(jnp.bfloat16)

    CH = D_OUT // 4
    for c in range(4):
        sl = pl.ds(c * CH, CH)
        base_c = jax.lax.dot_general(
            xh, wb_s[sl, :], (((1,), (1,)), ((), ())),
            preferred_element_type=jnp.float32)
        lora_c = jax.lax.dot_general(
            hw, abf_ref[:, D_IN + c * CH:D_IN + (c + 1) * CH], (((1,), (0,)), ((), ())),
            preferred_element_type=jnp.float32)
        out_ref[:, sl] = base_c + bb_ref[:, sl] + lora_c


def kernel(x, W_base, b_base, W_router, A, B):
    abf = jnp.concatenate(
        [A.reshape(E * RANK, D_IN),
         jnp.transpose(B, (0, 2, 1)).reshape(E * RANK, D_OUT)],
        axis=1).astype(jnp.bfloat16)
    bb = b_base.reshape(1, D_OUT)

    grid = (T // TILE,)
    return pl.pallas_call(
        _fused_kernel,
        grid=grid,
        in_specs=[
            pl.BlockSpec((TILE, D_IN), lambda i: (i, 0)),
            pl.BlockSpec((D_OUT, D_IN), lambda i: (0, 0)),
            pl.BlockSpec((1, D_OUT), lambda i: (0, 0)),
            pl.BlockSpec((E, D_IN), lambda i: (0, 0)),
            pl.BlockSpec((E * RANK, D_IN + D_OUT), lambda i: (0, 0)),
        ],
        out_specs=pl.BlockSpec((TILE, D_OUT), lambda i: (i, 0)),
        out_shape=jax.ShapeDtypeStruct((T, D_OUT), jnp.float32),
        scratch_shapes=[pltpu.VMEM((D_OUT, D_IN), jnp.bfloat16)],
        compiler_params=pltpu.CompilerParams(
            dimension_semantics=("arbitrary",),
        ),
    )(x, W_base, bb, W_router, abf)
